# baseline (speedup 1.0000x reference)
import functools

import jax
import jax.numpy as jnp
from jax import lax
from jax.experimental import pallas as pl
from jax.experimental.pallas import tpu as pltpu
from jax.experimental.pallas import tpu_sc as plsc

_SC_FRACTION_NUM = 1
_SC_FRACTION_DEN = 2


def _sc_gather(xf, idx, total, S, V, N, D):
    info = plsc.get_sparse_core_info()
    NC, NS, L = info.num_cores, info.num_subcores, info.num_lanes
    NW = NC * NS
    b_per_w = S // NW

    mesh = plsc.VectorSubcoreMesh(core_axis_name="c", subcore_axis_name="s")

    @functools.partial(
        pl.kernel,
        mesh=mesh,
        out_type=jax.ShapeDtypeStruct((total, D), jnp.float32),
        scratch_types=[
            pltpu.VMEM((b_per_w,), jnp.int32),
            pltpu.VMEM((b_per_w, D), jnp.float32),
            pltpu.SemaphoreType.DMA,
        ],
    )
    def gather_k(x_hbm, idx_hbm, out_hbm, idx_v, rows_v, sem):
        wid = lax.axis_index("s") * NC + lax.axis_index("c")
        base = wid * b_per_w
        pltpu.sync_copy(idx_hbm.at[pl.ds(base, b_per_w)], idx_v)
        for i in range(b_per_w // L):
            off = ((base + i * L) // N) * V
            idx_v[pl.ds(i * L, L)] = idx_v[pl.ds(i * L, L)] + off
        pltpu.async_copy(x_hbm.at[idx_v], rows_v, sem).wait()
        pltpu.sync_copy(rows_v, out_hbm.at[pl.ds(base, b_per_w)])

    return gather_k(xf, idx)


def _tc_fill(xf, idx_tc, partial, S, M, D, n_slots=8):
    def body(x_ref, idx_ref, partial_ref, out_ref, sem):
        def issue(i):
            r = idx_ref[i]
            return pltpu.make_async_copy(
                x_ref.at[r], out_ref.at[S + i], sem.at[lax.rem(i, n_slots)]
            )

        def step(i, carry):
            issue(i).start()
            return carry

        lax.fori_loop(0, M, step, 0)

        def drain(i, carry):
            issue(i).wait()
            return carry

        lax.fori_loop(0, M, drain, 0)

    return pl.pallas_call(
        body,
        grid=(),
        in_specs=[
            pl.BlockSpec(memory_space=pltpu.MemorySpace.HBM),
            pl.BlockSpec(memory_space=pltpu.SMEM),
            pl.BlockSpec(memory_space=pltpu.MemorySpace.HBM),
        ],
        out_specs=pl.BlockSpec(memory_space=pltpu.MemorySpace.HBM),
        out_shape=jax.ShapeDtypeStruct(partial.shape, partial.dtype),
        input_output_aliases={2: 0},
        scratch_shapes=[pltpu.SemaphoreType.DMA((n_slots,))],
    )(xf, idx_tc, partial)


def kernel(x, idx):
    B, V, D = x.shape
    _, N = idx.shape
    total = B * N
    S = total * _SC_FRACTION_NUM // _SC_FRACTION_DEN
    M = total - S

    xf = x.reshape(B * V, D)
    idxf = idx.reshape(total).astype(jnp.int32)
    partial = _sc_gather(xf, idxf, total, S, V, N, D)

    offs = (jnp.arange(total, dtype=jnp.int32) // N) * V
    idx_tc = (idxf + offs)[S:]
    out = _tc_fill(xf, idx_tc, partial, S, M, D)
    return out.reshape(B, N, D)


# speedup vs baseline: 5.9541x; 5.9541x over previous
import functools

import jax
import jax.numpy as jnp
from jax import lax
from jax.experimental import pallas as pl
from jax.experimental.pallas import tpu as pltpu
from jax.experimental.pallas import tpu_sc as plsc


def kernel(x, idx):
    B, V, D = x.shape
    _, N = idx.shape
    total = B * N

    info = plsc.get_sparse_core_info()
    NC, NS, L = info.num_cores, info.num_subcores, info.num_lanes
    NW = NC * NS
    b_per_w = total // NW

    mesh = plsc.VectorSubcoreMesh(core_axis_name="c", subcore_axis_name="s")

    @functools.partial(
        pl.kernel,
        mesh=mesh,
        out_type=jax.ShapeDtypeStruct((total, D), jnp.float32),
        scratch_types=[
            pltpu.VMEM((b_per_w,), jnp.int32),
            pltpu.VMEM((b_per_w, D), jnp.float32),
            pltpu.SemaphoreType.DMA,
        ],
    )
    def gather_k(x_hbm, idx_hbm, out_hbm, idx_v, rows_v, sem):
        wid = lax.axis_index("s") * NC + lax.axis_index("c")
        base = wid * b_per_w
        b = base // N
        pltpu.sync_copy(idx_hbm.at[b, pl.ds(base % N, b_per_w)], idx_v)
        for i in range(b_per_w // L):
            idx_v[pl.ds(i * L, L)] = idx_v[pl.ds(i * L, L)] + b * V
        pltpu.async_copy(x_hbm.at[idx_v], rows_v, sem).wait()
        pltpu.sync_copy(rows_v, out_hbm.at[pl.ds(base, b_per_w)])

    xf = x.reshape(B * V, D)
    out = gather_k(xf, idx.astype(jnp.int32))
    return out.reshape(B, N, D)
